# baseline (speedup 1.0000x reference)
import jax
import jax.numpy as jnp
from jax import lax
from jax.experimental import pallas as pl
from jax.experimental.pallas import tpu as pltpu

_BATCH = 1024
_CONN = 128
_LANES = 128
_NBUF = 32


def _strip_copy(a_ref, idx_ref, bufs, sems, j):
    col = idx_ref[j]
    off = pl.multiple_of((col // _LANES) * _LANES, _LANES)
    return pltpu.make_async_copy(
        a_ref.at[:, pl.ds(off, _LANES)],
        bufs.at[j % _NBUF],
        sems.at[j % _NBUF],
    )


def _body(idx_ref, a_ref, w_ref, sens_ref, thr_ref, o_ref, bufs, sems):
    for j in range(_NBUF):
        _strip_copy(a_ref, idx_ref, bufs, sems, j).start()
    lane = lax.broadcasted_iota(jnp.int32, (1, _LANES), 1)
    acc = jnp.zeros((_BATCH, _LANES), jnp.float32)
    for j in range(_CONN):
        _strip_copy(a_ref, idx_ref, bufs, sems, j).wait()
        sel = jnp.where(lane == idx_ref[j] % _LANES, w_ref[j], 0.0)
        acc = acc + bufs[j % _NBUF] * sel
        if j + _NBUF < _CONN:
            _strip_copy(a_ref, idx_ref, bufs, sems, j + _NBUF).start()
    z = jnp.sum(acc, axis=1)
    z = z * sens_ref[0] - thr_ref[0]
    o_ref[...] = 1.0 / (1.0 + jnp.exp(-z))


def kernel(x, all_activations, connection_weights, sensitivity, threshold,
           connection_indices):
    del x
    return pl.pallas_call(
        _body,
        grid_spec=pltpu.PrefetchScalarGridSpec(
            num_scalar_prefetch=1,
            in_specs=[
                pl.BlockSpec(memory_space=pl.ANY),
                pl.BlockSpec(memory_space=pltpu.SMEM),
                pl.BlockSpec(memory_space=pltpu.SMEM),
                pl.BlockSpec(memory_space=pltpu.SMEM),
            ],
            out_specs=pl.BlockSpec(memory_space=pltpu.VMEM),
            scratch_shapes=[
                pltpu.VMEM((_NBUF, _BATCH, _LANES), jnp.float32),
                pltpu.SemaphoreType.DMA((_NBUF,)),
            ],
        ),
        out_shape=jax.ShapeDtypeStruct((_BATCH,), jnp.float32),
    )(connection_indices, all_activations, connection_weights,
      sensitivity, threshold)


# speedup vs baseline: 34.4501x; 34.4501x over previous
import jax
import jax.numpy as jnp
from jax import lax
from jax.experimental import pallas as pl
from jax.experimental.pallas import tpu as pltpu

_BATCH = 1024
_CONN = 128
_SUB = 8
_NBUF = 16


def _slab_copy(at_ref, idx_ref, bufs, sems, j):
    base = pl.multiple_of((idx_ref[j] // _SUB) * _SUB, _SUB)
    return pltpu.make_async_copy(
        at_ref.at[pl.ds(base, _SUB), :],
        bufs.at[j % _NBUF],
        sems.at[j % _NBUF],
    )


def _body(idx_ref, at_ref, w_ref, sens_ref, thr_ref, o_ref, bufs, sems):
    for j in range(_NBUF):
        _slab_copy(at_ref, idx_ref, bufs, sems, j).start()
    sub = lax.broadcasted_iota(jnp.int32, (_SUB, 1), 0)
    acc = jnp.zeros((_SUB, _BATCH), jnp.float32)
    for j in range(_CONN):
        _slab_copy(at_ref, idx_ref, bufs, sems, j).wait()
        sel = jnp.where(sub == idx_ref[j] % _SUB, w_ref[j], 0.0)
        acc = acc + bufs[j % _NBUF] * sel
        if j + _NBUF < _CONN:
            _slab_copy(at_ref, idx_ref, bufs, sems, j + _NBUF).start()
    z = jnp.sum(acc, axis=0)
    z = z * sens_ref[0] - thr_ref[0]
    o_ref[...] = 1.0 / (1.0 + jnp.exp(-z))


def kernel(x, all_activations, connection_weights, sensitivity, threshold,
           connection_indices):
    del x
    return pl.pallas_call(
        _body,
        grid_spec=pltpu.PrefetchScalarGridSpec(
            num_scalar_prefetch=1,
            in_specs=[
                pl.BlockSpec(memory_space=pl.ANY),
                pl.BlockSpec(memory_space=pltpu.SMEM),
                pl.BlockSpec(memory_space=pltpu.SMEM),
                pl.BlockSpec(memory_space=pltpu.SMEM),
            ],
            out_specs=pl.BlockSpec(memory_space=pltpu.VMEM),
            scratch_shapes=[
                pltpu.VMEM((_NBUF, _SUB, _BATCH), jnp.float32),
                pltpu.SemaphoreType.DMA((_NBUF,)),
            ],
        ),
        out_shape=jax.ShapeDtypeStruct((_BATCH,), jnp.float32),
    )(connection_indices, all_activations.T, connection_weights,
      sensitivity, threshold)


# speedup vs baseline: 54.5233x; 1.5827x over previous
import jax
import jax.numpy as jnp
from jax import lax
from jax.experimental import pallas as pl
from jax.experimental.pallas import tpu as pltpu

_BATCH = 1024
_CONN = 128
_SUB = 8


def _slab_copy(at_ref, idx_ref, bufs, sems, j):
    base = pl.multiple_of((idx_ref[j] // _SUB) * _SUB, _SUB)
    return pltpu.make_async_copy(
        at_ref.at[pl.ds(base, _SUB), :],
        bufs.at[j],
        sems.at[j],
    )


def _body(idx_ref, at_ref, w_ref, sens_ref, thr_ref, o_ref, bufs, sems):
    for j in range(_CONN):
        _slab_copy(at_ref, idx_ref, bufs, sems, j).start()
    sub = lax.broadcasted_iota(jnp.int32, (_SUB, 1), 0)
    acc = jnp.zeros((_SUB, _BATCH), jnp.float32)
    for j in range(_CONN):
        _slab_copy(at_ref, idx_ref, bufs, sems, j).wait()
        sel = jnp.where(sub == idx_ref[j] % _SUB, w_ref[j], 0.0)
        acc = acc + bufs[j] * sel
    z = jnp.sum(acc, axis=0)
    z = z * sens_ref[0] - thr_ref[0]
    o_ref[...] = 1.0 / (1.0 + jnp.exp(-z))


def kernel(x, all_activations, connection_weights, sensitivity, threshold,
           connection_indices):
    del x
    return pl.pallas_call(
        _body,
        grid_spec=pltpu.PrefetchScalarGridSpec(
            num_scalar_prefetch=1,
            in_specs=[
                pl.BlockSpec(memory_space=pl.ANY),
                pl.BlockSpec(memory_space=pltpu.SMEM),
                pl.BlockSpec(memory_space=pltpu.SMEM),
                pl.BlockSpec(memory_space=pltpu.SMEM),
            ],
            out_specs=pl.BlockSpec(memory_space=pltpu.VMEM),
            scratch_shapes=[
                pltpu.VMEM((_CONN, _SUB, _BATCH), jnp.float32),
                pltpu.SemaphoreType.DMA((_CONN,)),
            ],
        ),
        out_shape=jax.ShapeDtypeStruct((_BATCH,), jnp.float32),
    )(connection_indices, all_activations.T, connection_weights,
      sensitivity, threshold)


# speedup vs baseline: 65.0801x; 1.1936x over previous
import jax
import jax.numpy as jnp
from jax import lax
from jax.experimental import pallas as pl
from jax.experimental.pallas import tpu as pltpu

_BATCH = 1024
_CONN = 128
_SUB = 8


def _row_copy(at_ref, idx_ref, bufs, sems, j):
    return pltpu.make_async_copy(
        at_ref.at[pl.ds(idx_ref[j], 1), :],
        bufs.at[j],
        sems.at[j],
    )


def _body(idx_ref, at_ref, w_ref, sens_ref, thr_ref, o_ref, bufs, sems):
    for j in range(_CONN):
        _row_copy(at_ref, idx_ref, bufs, sems, j).start()
    acc = jnp.zeros((1, _BATCH), jnp.float32)
    for j in range(_CONN):
        _row_copy(at_ref, idx_ref, bufs, sems, j).wait()
        acc = acc + bufs[j] * w_ref[j]
    z = acc[0]
    z = z * sens_ref[0] - thr_ref[0]
    o_ref[...] = 1.0 / (1.0 + jnp.exp(-z))


def kernel(x, all_activations, connection_weights, sensitivity, threshold,
           connection_indices):
    del x
    return pl.pallas_call(
        _body,
        grid_spec=pltpu.PrefetchScalarGridSpec(
            num_scalar_prefetch=1,
            in_specs=[
                pl.BlockSpec(memory_space=pl.ANY),
                pl.BlockSpec(memory_space=pltpu.SMEM),
                pl.BlockSpec(memory_space=pltpu.SMEM),
                pl.BlockSpec(memory_space=pltpu.SMEM),
            ],
            out_specs=pl.BlockSpec(memory_space=pltpu.VMEM),
            scratch_shapes=[
                pltpu.VMEM((_CONN, 1, _BATCH), jnp.float32),
                pltpu.SemaphoreType.DMA((_CONN,)),
            ],
        ),
        out_shape=jax.ShapeDtypeStruct((_BATCH,), jnp.float32),
    )(connection_indices, all_activations.T, connection_weights,
      sensitivity, threshold)
